# baseline (speedup 1.0000x reference)
import jax
import jax.numpy as jnp
from jax.experimental import pallas as pl
from jax.experimental.pallas import tpu as pltpu

_EPS_LN = 1e-5
_HB = 8


def _fused_block(x_ref, y_ref, lnw_ref, lnb_ref, w_ref, o_ref):
    z = x_ref[0] + y_ref[0]
    mean = jnp.mean(z, axis=-1, keepdims=True)
    zc = z - mean
    var = jnp.mean(zc * zc, axis=-1, keepdims=True)
    inv = jax.lax.rsqrt(var + _EPS_LN)
    normed = zc * (inv * lnw_ref[0]) + lnb_ref[0]
    acc = jax.lax.dot_general(
        w_ref[...], normed, (((1,), (0,)), ((), ())),
        preferred_element_type=jnp.float32)
    o_ref[0] = jnp.maximum(acc, 0.0)


def kernel(x, y, ln_weight, ln_bias, conv_weight):
    B, C, H, W = x.shape
    O = conv_weight.shape[0]
    lnw = ln_weight.reshape(1, 1, W)
    lnb = ln_bias.reshape(1, 1, W)
    grid = (B, H // _HB)
    return pl.pallas_call(
        _fused_block,
        grid=grid,
        in_specs=[
            pl.BlockSpec((1, C, _HB, W), lambda b, h: (b, 0, h, 0)),
            pl.BlockSpec((1, C, _HB, W), lambda b, h: (b, 0, h, 0)),
            pl.BlockSpec((1, 1, W), lambda b, h: (0, 0, 0)),
            pl.BlockSpec((1, 1, W), lambda b, h: (0, 0, 0)),
            pl.BlockSpec((O, C), lambda b, h: (0, 0)),
        ],
        out_specs=pl.BlockSpec((1, O, _HB, W), lambda b, h: (b, 0, h, 0)),
        out_shape=jax.ShapeDtypeStruct((B, O, H, W), jnp.float32),
        compiler_params=pltpu.CompilerParams(
            dimension_semantics=("parallel", "parallel"),
            vmem_limit_bytes=56 * 1024 * 1024,
        ),
    )(x, y, lnw, lnb, conv_weight)


# speedup vs baseline: 1.1869x; 1.1869x over previous
import jax
import jax.numpy as jnp
from jax.experimental import pallas as pl
from jax.experimental.pallas import tpu as pltpu

_EPS_LN = 1e-5
_HB = 32


def _fused_block(x_ref, y_ref, lnw_ref, lnb_ref, w_ref, o_ref):
    z = x_ref[0] + y_ref[0]
    mean = jnp.mean(z, axis=-1, keepdims=True)
    zc = z - mean
    var = jnp.mean(zc * zc, axis=-1, keepdims=True)
    inv = jax.lax.rsqrt(var + _EPS_LN)
    normed = zc * (inv * lnw_ref[0]) + lnb_ref[0]
    acc = jax.lax.dot_general(
        w_ref[...], normed, (((1,), (0,)), ((), ())),
        preferred_element_type=jnp.float32)
    o_ref[0] = jnp.maximum(acc, 0.0)


def kernel(x, y, ln_weight, ln_bias, conv_weight):
    B, C, H, W = x.shape
    O = conv_weight.shape[0]
    lnw = ln_weight.reshape(1, 1, W)
    lnb = ln_bias.reshape(1, 1, W)
    grid = (B, H // _HB)
    return pl.pallas_call(
        _fused_block,
        grid=grid,
        in_specs=[
            pl.BlockSpec((1, C, _HB, W), lambda b, h: (b, 0, h, 0)),
            pl.BlockSpec((1, C, _HB, W), lambda b, h: (b, 0, h, 0)),
            pl.BlockSpec((1, 1, W), lambda b, h: (0, 0, 0)),
            pl.BlockSpec((1, 1, W), lambda b, h: (0, 0, 0)),
            pl.BlockSpec((O, C), lambda b, h: (0, 0)),
        ],
        out_specs=pl.BlockSpec((1, O, _HB, W), lambda b, h: (b, 0, h, 0)),
        out_shape=jax.ShapeDtypeStruct((B, O, H, W), jnp.float32),
        compiler_params=pltpu.CompilerParams(
            dimension_semantics=("parallel", "parallel"),
            vmem_limit_bytes=63 * 1024 * 1024,
        ),
    )(x, y, lnw, lnb, conv_weight)
